# baseline (speedup 1.0000x reference)
import functools

import jax
import jax.numpy as jnp
from jax import lax
from jax.experimental import pallas as pl
from jax.experimental.pallas import tpu as pltpu
from jax.experimental.pallas import tpu_sc as plsc


@functools.lru_cache(maxsize=None)
def _build(B, F, NU1, NI1):
    info = plsc.get_sparse_core_info()
    NC, NS, L = info.num_cores, info.num_subcores, info.num_lanes
    NW = NC * NS
    assert B % NW == 0 and F == 2 * L
    BPW = B // NW
    G = BPW // L

    mesh = plsc.VectorSubcoreMesh(core_axis_name="c", subcore_axis_name="s")

    @functools.partial(
        pl.kernel,
        out_type=jax.ShapeDtypeStruct((B,), jnp.float32),
        mesh=mesh,
        scratch_types=[
            pltpu.VMEM((BPW,), jnp.int32),
            pltpu.VMEM((BPW,), jnp.int32),
            pltpu.VMEM((BPW, F), jnp.float32),
            pltpu.VMEM((BPW, F), jnp.float32),
            pltpu.VMEM((BPW,), jnp.float32),
            pltpu.VMEM((BPW,), jnp.float32),
            pltpu.VMEM((BPW,), jnp.float32),
            pltpu.SemaphoreType.DMA,
            pltpu.SemaphoreType.DMA,
            pltpu.SemaphoreType.DMA,
            pltpu.SemaphoreType.DMA,
        ],
        compiler_params=pltpu.CompilerParams(use_tc_tiling_on_sc=False),
    )
    def k(user_hbm, item_hbm, uf_hbm, itf_hbm, ub_hbm, ib_hbm, out_hbm,
          uidx_v, iidx_v, uf_v, itf_v, ub_v, ib_v, out_v,
          sem_uf, sem_if, sem_ub, sem_ib):
        wid = lax.axis_index("s") * NC + lax.axis_index("c")
        base = wid * BPW

        pltpu.sync_copy(user_hbm.at[pl.ds(base, BPW)], uidx_v)
        pltpu.sync_copy(item_hbm.at[pl.ds(base, BPW)], iidx_v)

        cu = pltpu.async_copy(uf_hbm.at[uidx_v], uf_v, sem_uf)
        ci = pltpu.async_copy(itf_hbm.at[iidx_v], itf_v, sem_if)
        cub = pltpu.async_copy(ub_hbm.at[uidx_v], ub_v, sem_ub)
        cib = pltpu.async_copy(ib_hbm.at[iidx_v], ib_v, sem_ib)
        cu.wait()
        ci.wait()
        cub.wait()
        cib.wait()

        lane = lax.broadcasted_iota(jnp.int32, (L,), 0)
        dnums = lax.GatherDimensionNumbers(
            offset_dims=(), collapsed_slice_dims=(0,), start_index_map=(0,))

        def perm(x, idx):
            return lax.gather(x, idx[:, None], dnums, (1,),
                              mode=lax.GatherScatterMode.PROMISE_IN_BOUNDS)

        bfly = [lane ^ sh for sh in (8, 4, 2, 1)]

        def group(g, carry):
            acc = jnp.zeros((L,), jnp.float32)
            for t in range(L):
                j = g * L + t
                p = (uf_v[j, pl.ds(0, L)] * itf_v[j, pl.ds(0, L)]
                     + uf_v[j, pl.ds(L, L)] * itf_v[j, pl.ds(L, L)])
                for bidx in bfly:
                    p = p + perm(p, bidx)
                acc = jnp.where(lane == t, p, acc)
            off = g * L
            out_v[pl.ds(off, L)] = (acc + ub_v[pl.ds(off, L)]
                                    + ib_v[pl.ds(off, L)])
            return carry

        lax.fori_loop(0, G, group, 0)

        pltpu.sync_copy(out_v, out_hbm.at[pl.ds(base, BPW)])

    return k


def kernel(user, item, user_factors, item_factors, user_bias, item_bias):
    B = user.shape[0]
    F = user_factors.shape[1]
    k = _build(B, F, user_factors.shape[0], item_factors.shape[0])
    return k(
        user.astype(jnp.int32),
        item.astype(jnp.int32),
        user_factors,
        item_factors,
        user_bias.reshape(-1),
        item_bias.reshape(-1),
    )
